# baseline (speedup 1.0000x reference)
import functools
import math

import jax
import jax.numpy as jnp
from jax import lax
from jax.experimental import pallas as pl
from jax.experimental.pallas import tpu as pltpu
from jax.experimental.pallas import tpu_sc as plsc

H = 128
R = 16
INV3 = 1.0 / math.sqrt(3.0)
INVH = 1.0 / math.sqrt(H)
SSCALE = 1.0 / 0.6

NC = 2
NS = 16
NW = NC * NS
CHUNK = 32

PAY = 12 * H
POOL = 5 * H
BATOM = 512


def _ssilu(x):
    return x * jax.nn.sigmoid(x) * SSCALE


def _dot(x, w):
    return lax.dot_general(x, w, (((1,), (1,)), ((), ())),
                           preferred_element_type=jnp.float32)


def _payload_kernel(s_ref, v_ref, w1b_ref, wv_ref, out_ref):
    for ax in range(3):
        w1b = w1b_ref[ax * H:(ax + 1) * H, :]
        out_ref[:, ax * H:(ax + 1) * H] = _dot(s_ref[:, ax * H:(ax + 1) * H], w1b)
        wv = wv_ref[ax * H:(ax + 1) * H, :]
        for d in range(3):
            c = ax * 3 + d
            out_ref[:, 3 * H + c * H:3 * H + (c + 1) * H] = _dot(
                v_ref[:, c * H:(c + 1) * H], wv)


def _build_payload(s_flat, v_flat, w1b, wv, G):
    BG = 256
    return pl.pallas_call(
        _payload_kernel,
        grid=(G // BG,),
        in_specs=[
            pl.BlockSpec((BG, 3 * H), lambda i: (i, 0)),
            pl.BlockSpec((BG, 9 * H), lambda i: (i, 0)),
            pl.BlockSpec((3 * H, H), lambda i: (0, 0)),
            pl.BlockSpec((3 * H, H), lambda i: (0, 0)),
        ],
        out_specs=pl.BlockSpec((BG, PAY), lambda i: (i, 0)),
        out_shape=jax.ShapeDtypeStruct((G, PAY), jnp.float32),
    )(s_flat, v_flat, w1b, wv)


def _make_gather(Npad):
    b_per_w = Npad // NW
    npair = b_per_w // (2 * CHUNK)
    mesh = plsc.VectorSubcoreMesh(core_axis_name="c", subcore_axis_name="s")

    @functools.partial(
        pl.kernel, mesh=mesh,
        out_type=jax.ShapeDtypeStruct((Npad, PAY), jnp.float32),
        scratch_types=[
            pltpu.VMEM((b_per_w,), jnp.int32),
            pltpu.VMEM((CHUNK, PAY), jnp.float32),
            pltpu.VMEM((CHUNK, PAY), jnp.float32),
            pltpu.SemaphoreType.DMA,
            pltpu.SemaphoreType.DMA,
            pltpu.SemaphoreType.DMA,
            pltpu.SemaphoreType.DMA,
        ],
    )
    def gather_k(table_hbm, idx_hbm, out_hbm, idx_all, rows0, rows1,
                 sg0, sg1, sw0, sw1):
        wid = lax.axis_index("s") * NC + lax.axis_index("c")
        base = wid * b_per_w
        pltpu.sync_copy(idx_hbm.at[pl.ds(base, b_per_w)], idx_all)

        def body(t, carry):
            l0 = (2 * t) * CHUNK
            l1 = l0 + CHUNK
            g0 = pltpu.async_copy(
                table_hbm.at[idx_all.at[pl.ds(l0, CHUNK)]], rows0, sg0)
            g1 = pltpu.async_copy(
                table_hbm.at[idx_all.at[pl.ds(l1, CHUNK)]], rows1, sg1)
            g0.wait()
            w0 = pltpu.async_copy(rows0, out_hbm.at[pl.ds(base + l0, CHUNK)], sw0)
            g1.wait()
            w1 = pltpu.async_copy(rows1, out_hbm.at[pl.ds(base + l1, CHUNK)], sw1)
            w0.wait()
            w1.wait()
            return carry

        lax.fori_loop(0, npair, body, 0)

    return gather_k


def _atom_kernel(N_real, B, G,
                 s_ref, v_ref, feat_ref, dir_ref, pay_ref, batch_ref,
                 wm1_ref, bm1_ref, wm2_ref, bm2_ref, wr_ref, br_ref,
                 w1a_ref, b1_ref, w2_ref, b2_ref, wv_ref,
                 out_ref):
    a_s = s_ref[...]
    a_v = [v_ref[:, d, :] for d in range(3)]
    for ax in range(3):
        wm1 = wm1_ref[ax * H:(ax + 1) * H, :]
        wm2 = wm2_ref[ax * 3 * H:(ax + 1) * 3 * H, :]
        wr = wr_ref[ax * 3 * H:(ax + 1) * 3 * H, :]
        atom_proj = _dot(_ssilu(_dot(a_s, wm1) + bm1_ref[ax:ax + 1, :]),
                         wm2) + bm2_ref[ax:ax + 1, :]
        edge_proj = _dot(feat_ref[:, ax, :], wr) + br_ref[ax:ax + 1, :]
        m = atom_proj * edge_proj * INV3
        m1 = m[:, :H]
        m2 = m[:, H:2 * H]
        a_s = a_s + m[:, 2 * H:]
        for d in range(3):
            edir = dir_ref[:, ax, d:d + 1]
            a_v[d] = (m1 * a_v[d] + m2 * edir) * INVH
        w1a = w1a_ref[ax * H:(ax + 1) * H, :]
        w2 = w2_ref[ax * H:(ax + 1) * H, :]
        sproj = pay_ref[:, ax * H:(ax + 1) * H]
        h = _ssilu(_dot(a_s, w1a) + sproj + b1_ref[ax:ax + 1, :])
        h = _ssilu(_dot(h, w2) + b2_ref[ax:ax + 1, :])
        a_s = a_s + h
        wv = wv_ref[ax * H:(ax + 1) * H, :]
        for d in range(3):
            c = ax * 3 + d
            vproj = pay_ref[:, 3 * H + c * H:3 * H + (c + 1) * H]
            a_v[d] = _dot(a_v[d], wv) + vproj + a_v[d]
    i = pl.program_id(0)
    rows = i * B + lax.broadcasted_iota(jnp.int32, (B, 1), 0)
    valid = rows < N_real
    parts = [jnp.where(valid, a_s, 0.0)]
    parts += [jnp.where(valid, a_v[d], 0.0) for d in range(3)]
    parts.append(jnp.broadcast_to(jnp.where(valid, 1.0, 0.0), (B, H)))
    vals = jnp.concatenate(parts, axis=1).astype(jnp.bfloat16)
    seg = lax.broadcasted_iota(jnp.int32, (G, B), 0)
    onehot = (seg == batch_ref[0]).astype(jnp.bfloat16)

    @pl.when(i == 0)
    def _():
        out_ref[...] = jnp.zeros_like(out_ref)

    out_ref[...] += lax.dot_general(
        onehot, vals, (((1,), (0,)), ((), ())),
        preferred_element_type=jnp.float32)


def _run_atoms(N_real, Npad, G, s_pad, v_pad, feat_pad, dir_pad, payload,
               batch3, wm1, bm1, wm2, bm2, wr, br, w1a, b1, w2, b2, wv):
    B = BATOM
    full = lambda r, c: pl.BlockSpec((r, c), lambda i: (0, 0))
    return pl.pallas_call(
        functools.partial(_atom_kernel, N_real, B, G),
        grid=((N_real + B - 1) // B,),
        in_specs=[
            pl.BlockSpec((B, H), lambda i: (i, 0)),
            pl.BlockSpec((B, 3, H), lambda i: (i, 0, 0)),
            pl.BlockSpec((B, 3, R), lambda i: (i, 0, 0)),
            pl.BlockSpec((B, 3, 3), lambda i: (i, 0, 0)),
            pl.BlockSpec((B, PAY), lambda i: (i, 0)),
            pl.BlockSpec((1, 1, B), lambda i: (i, 0, 0)),
            full(3 * H, H), full(3, H),
            full(9 * H, H), full(3, 3 * H),
            full(9 * H, R), full(3, 3 * H),
            full(3 * H, H), full(3, H),
            full(3 * H, H), full(3, H),
            full(3 * H, H),
        ],
        out_specs=pl.BlockSpec((G, POOL), lambda i: (0, 0)),
        out_shape=jax.ShapeDtypeStruct((G, POOL), jnp.float32),
        compiler_params=pltpu.CompilerParams(
            dimension_semantics=("arbitrary",)),
    )(s_pad, v_pad, feat_pad, dir_pad, payload, batch3,
      wm1, bm1, wm2, bm2, wr, br, w1a, b1, w2, b2, wv)


def _axis_kernel(part_ref, s_ref, v_ref,
                 wa1p_ref, wa1s_ref, ba1_ref, wa2_ref, ba2_ref, wav_ref,
                 wsv_ref, ws1s_ref, ws1n_ref, bs1_ref, ws2_ref, bs2_ref,
                 wd_ref, mix_ref, out_ref):
    pooled = part_ref[...]
    cnt = jnp.maximum(pooled[:, 4 * H:4 * H + 1], 1.0)
    inv_cnt = 1.0 / cnt
    pooled_s = pooled[:, 0:H] * inv_cnt
    pooled_v = [pooled[:, H + d * H:H + (d + 1) * H] * inv_cnt for d in range(3)]

    s_in = [s_ref[:, k * H:(k + 1) * H] for k in range(3)]
    v_in = [[v_ref[:, (k * 3 + d) * H:(k * 3 + d + 1) * H] for d in range(3)]
            for k in range(3)]
    s_mix = []
    v_mix = []
    for ax in range(3):
        acc = mix_ref[3 + ax, 0] + mix_ref[ax, 0] * s_in[0] \
            + mix_ref[ax, 1] * s_in[1] + mix_ref[ax, 2] * s_in[2]
        s_mix.append(acc)
        v_mix.append([mix_ref[6 + ax, 0] * v_in[0][d]
                      + mix_ref[6 + ax, 1] * v_in[1][d]
                      + mix_ref[6 + ax, 2] * v_in[2][d] for d in range(3)])

    s_new = []
    v_new = []
    for ax in range(3):
        wa1p = wa1p_ref[ax * H:(ax + 1) * H, :]
        wa1s = wa1s_ref[ax * H:(ax + 1) * H, :]
        wa2 = wa2_ref[ax * H:(ax + 1) * H, :]
        wav = wav_ref[ax * H:(ax + 1) * H, :]
        s_ax = s_mix[ax]
        h = _ssilu(_dot(pooled_s, wa1p) + _dot(s_ax, wa1s)
                   + ba1_ref[ax:ax + 1, :])
        d_s = _ssilu(_dot(h, wa2) + ba2_ref[ax:ax + 1, :])
        s_ax = s_ax + d_s
        v_ax = [v_mix[ax][d] + _dot(pooled_v[d] + v_mix[ax][d], wav)
                for d in range(3)]
        wsv = wsv_ref[ax * 2 * H:(ax + 1) * 2 * H, :]
        vv = [_dot(v_ax[d], wsv) for d in range(3)]
        v1 = [vv[d][:, :H] for d in range(3)]
        v2 = [vv[d][:, H:] for d in range(3)]
        norm = jnp.sqrt(v2[0] * v2[0] + v2[1] * v2[1] + v2[2] * v2[2] + 1e-8)
        ws1s = ws1s_ref[ax * H:(ax + 1) * H, :]
        ws1n = ws1n_ref[ax * H:(ax + 1) * H, :]
        ws2 = ws2_ref[ax * 3 * H:(ax + 1) * 3 * H, :]
        sh = _ssilu(_dot(s_ax, ws1s) + _dot(norm, ws1n)
                    + bs1_ref[ax:ax + 1, :])
        sh = _dot(sh, ws2) + bs2_ref[ax:ax + 1, :]
        s1 = sh[:, :H]
        s2 = sh[:, H:2 * H]
        gate = jnp.tanh(sh[:, 2 * H:])
        s_new.append(s2 + s_ax * gate)
        v_new.append([s1 * v1[d] + v_ax[d] for d in range(3)])

    cols = [None] * 9
    for ax in range(3):
        wd = wd_ref[ax:ax + 1, :]
        for d in range(3):
            vm = mix_ref[6 + ax, 0] * v_new[0][d] \
               + mix_ref[6 + ax, 1] * v_new[1][d] \
               + mix_ref[6 + ax, 2] * v_new[2][d]
            cols[d * 3 + ax] = _dot(vm, wd)
    out_ref[...] = jnp.concatenate(cols, axis=1)


def _run_axis(G, partials, s_flat, v_flat,
              wa1p, wa1s, ba1, wa2, ba2, wav, wsv, ws1s, ws1n, bs1,
              ws2, bs2, wd, mix):
    BG = 256
    full = lambda r, c: pl.BlockSpec((r, c), lambda i: (0, 0))
    return pl.pallas_call(
        _axis_kernel,
        grid=(G // BG,),
        in_specs=[
            pl.BlockSpec((BG, POOL), lambda i: (i, 0)),
            pl.BlockSpec((BG, 3 * H), lambda i: (i, 0)),
            pl.BlockSpec((BG, 9 * H), lambda i: (i, 0)),
            full(3 * H, H), full(3 * H, H), full(3, H),
            full(3 * H, H), full(3, H),
            full(3 * H, H),
            full(6 * H, H),
            full(3 * H, H), full(3 * H, H), full(3, H),
            full(9 * H, H), full(3, 3 * H),
            full(3, H),
            pl.BlockSpec(memory_space=pltpu.SMEM),
        ],
        out_specs=pl.BlockSpec((BG, 9), lambda i: (i, 0)),
        out_shape=jax.ShapeDtypeStruct((G, 9), jnp.float32),
    )(partials, s_flat, v_flat,
      wa1p, wa1s, ba1, wa2, ba2, wav, wsv, ws1s, ws1n, bs1, ws2, bs2, wd, mix)


def kernel(atom_scalar, axis_scalar_state, atom_vector, axis_vector_state,
           lattice_feat, lattice_udiff, batch,
           W_msg1, b_msg1, W_msg2, b_msg2, W_rbf, b_rbf,
           W_x2a_s1, b_x2a_s1, W_x2a_s2, b_x2a_s2, W_x2a_v,
           W_a2x_s1, b_a2x_s1, W_a2x_s2, b_a2x_s2, W_a2x_v,
           W_svec, W_ss1, b_ss1, W_ss2, b_ss2, W_delta,
           W_mix_s, b_mix_s, W_mix_v):
    N = atom_scalar.shape[0]
    G = axis_scalar_state.shape[0]
    align = CHUNK * NW
    Npad = ((N + align - 1) // align) * align

    f32 = jnp.float32
    s_flat = axis_scalar_state.reshape(G, 3 * H)
    v_flat = axis_vector_state.reshape(G, 9 * H)

    w1b = W_x2a_s1[:, :, H:].reshape(3 * H, H)
    wv = W_x2a_v.reshape(3 * H, H)
    table = _build_payload(s_flat, v_flat, w1b, wv, G)

    batch_pad = jnp.pad(batch.astype(jnp.int32), (0, Npad - N))
    payload = _make_gather(Npad)(table, batch_pad)

    batch3 = batch_pad.reshape(Npad // BATOM, 1, BATOM)
    pooled = _run_atoms(
        N, Npad, G, atom_scalar, atom_vector, lattice_feat, lattice_udiff,
        payload, batch3,
        W_msg1.reshape(3 * H, H), b_msg1,
        W_msg2.reshape(9 * H, H), b_msg2,
        W_rbf.reshape(9 * H, R), b_rbf,
        W_x2a_s1[:, :, :H].reshape(3 * H, H), b_x2a_s1,
        W_x2a_s2.reshape(3 * H, H), b_x2a_s2,
        W_x2a_v.reshape(3 * H, H))

    wsv = W_svec.reshape(6 * H, H)
    mix = jnp.concatenate(
        [W_mix_s, b_mix_s[:, None] * jnp.ones((3, 3), f32), W_mix_v], axis=0)
    out9 = _run_axis(
        G, pooled, s_flat, v_flat,
        W_a2x_s1[:, :, :H].reshape(3 * H, H),
        W_a2x_s1[:, :, H:].reshape(3 * H, H), b_a2x_s1,
        W_a2x_s2.reshape(3 * H, H), b_a2x_s2,
        W_a2x_v.reshape(3 * H, H),
        wsv,
        W_ss1[:, :, :H].reshape(3 * H, H),
        W_ss1[:, :, H:].reshape(3 * H, H), b_ss1,
        W_ss2.reshape(9 * H, H), b_ss2,
        W_delta.reshape(3, H), mix)
    return out9.reshape(G, 3, 3)


# speedup vs baseline: 1.3052x; 1.3052x over previous
import functools
import math

import jax
import jax.numpy as jnp
from jax import lax
from jax.experimental import pallas as pl
from jax.experimental.pallas import tpu as pltpu
from jax.experimental.pallas import tpu_sc as plsc

H = 128
R = 16
INV3 = 1.0 / math.sqrt(3.0)
INVH = 1.0 / math.sqrt(H)
SSCALE = 1.0 / 0.6

NC = 2
NS = 16
NW = NC * NS
CHUNK = 32

PAY = 12 * H
POOL = 5 * H
BATOM = 512


def _ssilu(x):
    return x * jax.nn.sigmoid(x) * SSCALE


def _dot(x, w):
    return lax.dot_general(x, w, (((1,), (1,)), ((), ())),
                           preferred_element_type=jnp.float32)


def _payload_kernel(s_ref, v_ref, w1b_ref, wv_ref, out_ref):
    for ax in range(3):
        w1b = w1b_ref[ax * H:(ax + 1) * H, :]
        out_ref[:, ax * H:(ax + 1) * H] = _dot(s_ref[:, ax * H:(ax + 1) * H], w1b)
        wv = wv_ref[ax * H:(ax + 1) * H, :]
        for d in range(3):
            c = ax * 3 + d
            out_ref[:, 3 * H + c * H:3 * H + (c + 1) * H] = _dot(
                v_ref[:, c * H:(c + 1) * H], wv)


def _build_payload(s_flat, v_flat, w1b, wv, G):
    BG = 256
    return pl.pallas_call(
        _payload_kernel,
        grid=(G // BG,),
        in_specs=[
            pl.BlockSpec((BG, 3 * H), lambda i: (i, 0)),
            pl.BlockSpec((BG, 9 * H), lambda i: (i, 0)),
            pl.BlockSpec((3 * H, H), lambda i: (0, 0)),
            pl.BlockSpec((3 * H, H), lambda i: (0, 0)),
        ],
        out_specs=pl.BlockSpec((BG, PAY), lambda i: (i, 0)),
        out_shape=jax.ShapeDtypeStruct((G, PAY), jnp.float32),
    )(s_flat, v_flat, w1b, wv)


def _make_gather(Npad):
    b_per_w = Npad // NW
    npair = b_per_w // (2 * CHUNK)
    mesh = plsc.VectorSubcoreMesh(core_axis_name="c", subcore_axis_name="s")

    @functools.partial(
        pl.kernel, mesh=mesh,
        out_type=jax.ShapeDtypeStruct((Npad, PAY), jnp.float32),
        scratch_types=[
            pltpu.VMEM((b_per_w,), jnp.int32),
            pltpu.VMEM((CHUNK, PAY), jnp.float32),
            pltpu.VMEM((CHUNK, PAY), jnp.float32),
            pltpu.SemaphoreType.DMA,
            pltpu.SemaphoreType.DMA,
            pltpu.SemaphoreType.DMA,
            pltpu.SemaphoreType.DMA,
        ],
    )
    def gather_k(table_hbm, idx_hbm, out_hbm, idx_all, rows0, rows1,
                 sg0, sg1, sw0, sw1):
        wid = lax.axis_index("s") * NC + lax.axis_index("c")
        base = wid * b_per_w
        pltpu.sync_copy(idx_hbm.at[pl.ds(base, b_per_w)], idx_all)

        def body(t, carry):
            l0 = (2 * t) * CHUNK
            l1 = l0 + CHUNK
            g0 = pltpu.async_copy(
                table_hbm.at[idx_all.at[pl.ds(l0, CHUNK)]], rows0, sg0)
            g1 = pltpu.async_copy(
                table_hbm.at[idx_all.at[pl.ds(l1, CHUNK)]], rows1, sg1)
            g0.wait()
            w0 = pltpu.async_copy(rows0, out_hbm.at[pl.ds(base + l0, CHUNK)], sw0)
            g1.wait()
            w1 = pltpu.async_copy(rows1, out_hbm.at[pl.ds(base + l1, CHUNK)], sw1)
            w0.wait()
            w1.wait()
            return carry

        lax.fori_loop(0, npair, body, 0)

    return gather_k


def _atom_kernel(N_real, B, G,
                 s_ref, v_ref, feat_ref, dir_ref, pay_ref, batch_ref,
                 wm1_ref, bm1_ref, wm2_ref, bm2_ref, wr_ref, br_ref,
                 w1a_ref, b1_ref, w2_ref, b2_ref, wv_ref,
                 out_ref):
    a_s = s_ref[...]
    a_v = [v_ref[:, d * H:(d + 1) * H] for d in range(3)]
    for ax in range(3):
        wm1 = wm1_ref[ax * H:(ax + 1) * H, :]
        wm2 = wm2_ref[ax * 3 * H:(ax + 1) * 3 * H, :]
        wr = wr_ref[ax * 3 * H:(ax + 1) * 3 * H, :]
        atom_proj = _dot(_ssilu(_dot(a_s, wm1) + bm1_ref[ax:ax + 1, :]),
                         wm2) + bm2_ref[ax:ax + 1, :]
        edge_proj = _dot(feat_ref[:, ax * R:(ax + 1) * R], wr) + br_ref[ax:ax + 1, :]
        m = atom_proj * edge_proj * INV3
        m1 = m[:, :H]
        m2 = m[:, H:2 * H]
        a_s = a_s + m[:, 2 * H:]
        for d in range(3):
            edir = dir_ref[:, ax * 3 + d:ax * 3 + d + 1]
            a_v[d] = (m1 * a_v[d] + m2 * edir) * INVH
        w1a = w1a_ref[ax * H:(ax + 1) * H, :]
        w2 = w2_ref[ax * H:(ax + 1) * H, :]
        sproj = pay_ref[:, ax * H:(ax + 1) * H]
        h = _ssilu(_dot(a_s, w1a) + sproj + b1_ref[ax:ax + 1, :])
        h = _ssilu(_dot(h, w2) + b2_ref[ax:ax + 1, :])
        a_s = a_s + h
        wv = wv_ref[ax * H:(ax + 1) * H, :]
        for d in range(3):
            c = ax * 3 + d
            vproj = pay_ref[:, 3 * H + c * H:3 * H + (c + 1) * H]
            a_v[d] = _dot(a_v[d], wv) + vproj + a_v[d]
    i = pl.program_id(0)
    rows = i * B + lax.broadcasted_iota(jnp.int32, (B, 1), 0)
    valid = rows < N_real
    parts = [jnp.where(valid, a_s, 0.0)]
    parts += [jnp.where(valid, a_v[d], 0.0) for d in range(3)]
    parts.append(jnp.broadcast_to(jnp.where(valid, 1.0, 0.0), (B, H)))
    vals = jnp.concatenate(parts, axis=1).astype(jnp.bfloat16)
    seg = lax.broadcasted_iota(jnp.int32, (G, B), 0)
    onehot = (seg == batch_ref[0]).astype(jnp.bfloat16)

    @pl.when(i == 0)
    def _():
        out_ref[...] = jnp.zeros_like(out_ref)

    out_ref[...] += lax.dot_general(
        onehot, vals, (((1,), (0,)), ((), ())),
        preferred_element_type=jnp.float32)


def _run_atoms(N_real, Npad, G, s_pad, v_pad, feat_pad, dir_pad, payload,
               batch3, wm1, bm1, wm2, bm2, wr, br, w1a, b1, w2, b2, wv):
    B = BATOM
    full = lambda r, c: pl.BlockSpec((r, c), lambda i: (0, 0))
    return pl.pallas_call(
        functools.partial(_atom_kernel, N_real, B, G),
        grid=((N_real + B - 1) // B,),
        in_specs=[
            pl.BlockSpec((B, H), lambda i: (i, 0)),
            pl.BlockSpec((B, 3 * H), lambda i: (i, 0)),
            pl.BlockSpec((B, 3 * R), lambda i: (i, 0)),
            pl.BlockSpec((B, 9), lambda i: (i, 0)),
            pl.BlockSpec((B, PAY), lambda i: (i, 0)),
            pl.BlockSpec((1, 1, B), lambda i: (i, 0, 0)),
            full(3 * H, H), full(3, H),
            full(9 * H, H), full(3, 3 * H),
            full(9 * H, R), full(3, 3 * H),
            full(3 * H, H), full(3, H),
            full(3 * H, H), full(3, H),
            full(3 * H, H),
        ],
        out_specs=pl.BlockSpec((G, POOL), lambda i: (0, 0)),
        out_shape=jax.ShapeDtypeStruct((G, POOL), jnp.float32),
        compiler_params=pltpu.CompilerParams(
            dimension_semantics=("arbitrary",)),
    )(s_pad, v_pad, feat_pad, dir_pad, payload, batch3,
      wm1, bm1, wm2, bm2, wr, br, w1a, b1, w2, b2, wv)


def _axis_kernel(part_ref, s_ref, v_ref,
                 wa1p_ref, wa1s_ref, ba1_ref, wa2_ref, ba2_ref, wav_ref,
                 wsv_ref, ws1s_ref, ws1n_ref, bs1_ref, ws2_ref, bs2_ref,
                 wd_ref, mix_ref, out_ref):
    pooled = part_ref[...]
    cnt = jnp.maximum(pooled[:, 4 * H:4 * H + 1], 1.0)
    inv_cnt = 1.0 / cnt
    pooled_s = pooled[:, 0:H] * inv_cnt
    pooled_v = [pooled[:, H + d * H:H + (d + 1) * H] * inv_cnt for d in range(3)]

    s_in = [s_ref[:, k * H:(k + 1) * H] for k in range(3)]
    v_in = [[v_ref[:, (k * 3 + d) * H:(k * 3 + d + 1) * H] for d in range(3)]
            for k in range(3)]
    s_mix = []
    v_mix = []
    for ax in range(3):
        acc = mix_ref[3 + ax, 0] + mix_ref[ax, 0] * s_in[0] \
            + mix_ref[ax, 1] * s_in[1] + mix_ref[ax, 2] * s_in[2]
        s_mix.append(acc)
        v_mix.append([mix_ref[6 + ax, 0] * v_in[0][d]
                      + mix_ref[6 + ax, 1] * v_in[1][d]
                      + mix_ref[6 + ax, 2] * v_in[2][d] for d in range(3)])

    s_new = []
    v_new = []
    for ax in range(3):
        wa1p = wa1p_ref[ax * H:(ax + 1) * H, :]
        wa1s = wa1s_ref[ax * H:(ax + 1) * H, :]
        wa2 = wa2_ref[ax * H:(ax + 1) * H, :]
        wav = wav_ref[ax * H:(ax + 1) * H, :]
        s_ax = s_mix[ax]
        h = _ssilu(_dot(pooled_s, wa1p) + _dot(s_ax, wa1s)
                   + ba1_ref[ax:ax + 1, :])
        d_s = _ssilu(_dot(h, wa2) + ba2_ref[ax:ax + 1, :])
        s_ax = s_ax + d_s
        v_ax = [v_mix[ax][d] + _dot(pooled_v[d] + v_mix[ax][d], wav)
                for d in range(3)]
        wsv = wsv_ref[ax * 2 * H:(ax + 1) * 2 * H, :]
        vv = [_dot(v_ax[d], wsv) for d in range(3)]
        v1 = [vv[d][:, :H] for d in range(3)]
        v2 = [vv[d][:, H:] for d in range(3)]
        norm = jnp.sqrt(v2[0] * v2[0] + v2[1] * v2[1] + v2[2] * v2[2] + 1e-8)
        ws1s = ws1s_ref[ax * H:(ax + 1) * H, :]
        ws1n = ws1n_ref[ax * H:(ax + 1) * H, :]
        ws2 = ws2_ref[ax * 3 * H:(ax + 1) * 3 * H, :]
        sh = _ssilu(_dot(s_ax, ws1s) + _dot(norm, ws1n)
                    + bs1_ref[ax:ax + 1, :])
        sh = _dot(sh, ws2) + bs2_ref[ax:ax + 1, :]
        s1 = sh[:, :H]
        s2 = sh[:, H:2 * H]
        gate = jnp.tanh(sh[:, 2 * H:])
        s_new.append(s2 + s_ax * gate)
        v_new.append([s1 * v1[d] + v_ax[d] for d in range(3)])

    cols = [None] * 9
    for ax in range(3):
        wd = wd_ref[ax:ax + 1, :]
        for d in range(3):
            vm = mix_ref[6 + ax, 0] * v_new[0][d] \
               + mix_ref[6 + ax, 1] * v_new[1][d] \
               + mix_ref[6 + ax, 2] * v_new[2][d]
            cols[d * 3 + ax] = _dot(vm, wd)
    out_ref[...] = jnp.concatenate(cols, axis=1)


def _run_axis(G, partials, s_flat, v_flat,
              wa1p, wa1s, ba1, wa2, ba2, wav, wsv, ws1s, ws1n, bs1,
              ws2, bs2, wd, mix):
    BG = 256
    full = lambda r, c: pl.BlockSpec((r, c), lambda i: (0, 0))
    return pl.pallas_call(
        _axis_kernel,
        grid=(G // BG,),
        in_specs=[
            pl.BlockSpec((BG, POOL), lambda i: (i, 0)),
            pl.BlockSpec((BG, 3 * H), lambda i: (i, 0)),
            pl.BlockSpec((BG, 9 * H), lambda i: (i, 0)),
            full(3 * H, H), full(3 * H, H), full(3, H),
            full(3 * H, H), full(3, H),
            full(3 * H, H),
            full(6 * H, H),
            full(3 * H, H), full(3 * H, H), full(3, H),
            full(9 * H, H), full(3, 3 * H),
            full(3, H),
            pl.BlockSpec(memory_space=pltpu.SMEM),
        ],
        out_specs=pl.BlockSpec((BG, 9), lambda i: (i, 0)),
        out_shape=jax.ShapeDtypeStruct((G, 9), jnp.float32),
    )(partials, s_flat, v_flat,
      wa1p, wa1s, ba1, wa2, ba2, wav, wsv, ws1s, ws1n, bs1, ws2, bs2, wd, mix)


def kernel(atom_scalar, axis_scalar_state, atom_vector, axis_vector_state,
           lattice_feat, lattice_udiff, batch,
           W_msg1, b_msg1, W_msg2, b_msg2, W_rbf, b_rbf,
           W_x2a_s1, b_x2a_s1, W_x2a_s2, b_x2a_s2, W_x2a_v,
           W_a2x_s1, b_a2x_s1, W_a2x_s2, b_a2x_s2, W_a2x_v,
           W_svec, W_ss1, b_ss1, W_ss2, b_ss2, W_delta,
           W_mix_s, b_mix_s, W_mix_v):
    N = atom_scalar.shape[0]
    G = axis_scalar_state.shape[0]
    align = CHUNK * NW
    Npad = ((N + align - 1) // align) * align

    f32 = jnp.float32
    s_flat = axis_scalar_state.reshape(G, 3 * H)
    v_flat = axis_vector_state.reshape(G, 9 * H)

    w1b = W_x2a_s1[:, :, H:].reshape(3 * H, H)
    wv = W_x2a_v.reshape(3 * H, H)
    table = _build_payload(s_flat, v_flat, w1b, wv, G)

    batch_pad = jnp.pad(batch.astype(jnp.int32), (0, Npad - N))
    payload = _make_gather(Npad)(table, batch_pad)

    batch3 = batch_pad.reshape(Npad // BATOM, 1, BATOM)
    pooled = _run_atoms(
        N, Npad, G, atom_scalar, atom_vector.reshape(N, 3 * H),
        lattice_feat.reshape(N, 3 * R), lattice_udiff.reshape(N, 9),
        payload, batch3,
        W_msg1.reshape(3 * H, H), b_msg1,
        W_msg2.reshape(9 * H, H), b_msg2,
        W_rbf.reshape(9 * H, R), b_rbf,
        W_x2a_s1[:, :, :H].reshape(3 * H, H), b_x2a_s1,
        W_x2a_s2.reshape(3 * H, H), b_x2a_s2,
        W_x2a_v.reshape(3 * H, H))

    wsv = W_svec.reshape(6 * H, H)
    mix = jnp.concatenate(
        [W_mix_s, b_mix_s[:, None] * jnp.ones((3, 3), f32), W_mix_v], axis=0)
    out9 = _run_axis(
        G, pooled, s_flat, v_flat,
        W_a2x_s1[:, :, :H].reshape(3 * H, H),
        W_a2x_s1[:, :, H:].reshape(3 * H, H), b_a2x_s1,
        W_a2x_s2.reshape(3 * H, H), b_a2x_s2,
        W_a2x_v.reshape(3 * H, H),
        wsv,
        W_ss1[:, :, :H].reshape(3 * H, H),
        W_ss1[:, :, H:].reshape(3 * H, H), b_ss1,
        W_ss2.reshape(9 * H, H), b_ss2,
        W_delta.reshape(3, H), mix)
    return out9.reshape(G, 3, 3)
